# baseline (speedup 1.0000x reference)
import functools

import jax
import jax.numpy as jnp
from jax import lax
from jax.experimental import pallas as pl
from jax.experimental.pallas import tpu as pltpu
from jax.experimental.pallas import tpu_sc as plsc

NPAD = 1_048_576
NW = 32
PER_W = NPAD // NW
CHUNK = 2048
NCHUNK = PER_W // CHUNK
LANES = 16
VECS = CHUNK // LANES
GRID = 4096
NCELL = (GRID - 2) * GRID + GRID - 2 + 2
QROWS = NCELL // 2


def _body(pts_hbm, zq_hbm, out_hbm,
          pb0, pb1, ib0, ib1, vq0, vq1, ob0, ob1, sem0, sem1):
    pb = (pb0, pb1)
    ib = (ib0, ib1)
    vq = (vq0, vq1)
    ob = (ob0, ob1)
    sem = (sem0, sem1)

    c = lax.axis_index("c")
    s = lax.axis_index("s")
    base = (s * 2 + c) * PER_W

    lane = lax.iota(jnp.int32, LANES)
    lane2 = lane * 2

    def load_pts(p, off):
        pltpu.sync_copy(pts_hbm.at[pl.ds(off * 2, CHUNK * 2)], pb[p])

    def point_xy(p, i):
        xv = plsc.load_gather(pb[p], [i * (2 * LANES) + lane2])
        yv = plsc.load_gather(pb[p], [i * (2 * LANES) + lane2 + 1])
        return xv, yv

    def cell_xy(xv, yv):
        ix = jnp.clip(xv.astype(jnp.int32), 0, GRID - 2)
        iy = jnp.clip(yv.astype(jnp.int32), 0, GRID - 2)
        return ix, iy

    def compute_idx(p):
        def body(i, carry):
            xv, yv = point_xy(p, i)
            ix, iy = cell_xy(xv, yv)
            cell = ix * GRID + iy
            ib[p][pl.ds(i * LANES, LANES)] = lax.shift_right_logical(cell, 1)
            return carry

        lax.fori_loop(0, VECS, body, 0)

    def fire(p):
        return pltpu.async_copy(zq_hbm.at[ib[p]], vq[p], sem[p])

    def mix(p, off):
        def body(i, carry):
            xv, yv = point_xy(p, i)
            ix, iy = cell_xy(xv, yv)
            wx = xv - ix.astype(jnp.float32)
            wy = yv - iy.astype(jnp.float32)
            row = i * LANES + lane
            cb = (iy & 1) * 4
            z00 = plsc.load_gather(vq[p], [row, cb])
            z01 = plsc.load_gather(vq[p], [row, cb + 1])
            z10 = plsc.load_gather(vq[p], [row, cb + 2])
            z11 = plsc.load_gather(vq[p], [row, cb + 3])
            a = z00 + (z01 - z00) * wy
            b = z10 + (z11 - z10) * wy
            ob[p][pl.ds(i * LANES, LANES)] = a + (b - a) * wx
            return carry

        lax.fori_loop(0, VECS, body, 0)
        pltpu.sync_copy(ob[p], out_hbm.at[pl.ds(off, CHUNK)])

    load_pts(0, base)
    compute_idx(0)
    cp = fire(0)
    for ci in range(NCHUNK):
        p = ci & 1
        q = p ^ 1
        nxt = None
        if ci + 1 < NCHUNK:
            load_pts(q, base + (ci + 1) * CHUNK)
            compute_idx(q)
            nxt = fire(q)
        cp.wait()
        mix(p, base + ci * CHUNK)
        cp = nxt


_interp = functools.partial(
    pl.kernel,
    out_type=jax.ShapeDtypeStruct((NPAD,), jnp.float32),
    mesh=plsc.VectorSubcoreMesh(core_axis_name="c", subcore_axis_name="s"),
    compiler_params=pltpu.CompilerParams(
        needs_layout_passes=False, use_tc_tiling_on_sc=False),
    scratch_types=[
        pltpu.VMEM((CHUNK * 2,), jnp.float32),
        pltpu.VMEM((CHUNK * 2,), jnp.float32),
        pltpu.VMEM((CHUNK,), jnp.int32),
        pltpu.VMEM((CHUNK,), jnp.int32),
        pltpu.VMEM((CHUNK, 8), jnp.float32),
        pltpu.VMEM((CHUNK, 8), jnp.float32),
        pltpu.VMEM((CHUNK,), jnp.float32),
        pltpu.VMEM((CHUNK,), jnp.float32),
        pltpu.SemaphoreType.DMA,
        pltpu.SemaphoreType.DMA,
    ],
)(_body)


def kernel(points_to_interpolate, xs, ys, zs, repeats=1):
    n = points_to_interpolate.shape[0]
    pts = jnp.pad(points_to_interpolate, ((0, NPAD - n), (0, 0)))
    zf = jnp.concatenate([zs.reshape(-1), jnp.zeros((1,), jnp.float32)])
    zq = jnp.stack(
        [zf[o:o + NCELL] for o in (0, 1, GRID, GRID + 1)], axis=1)
    out = _interp(pts.reshape(-1), zq.reshape(QROWS, 8))
    return out[:n]
